# Initial kernel scaffold; baseline (speedup 1.0000x reference)
import jax
import jax.numpy as jnp
from jax.experimental import pallas as pl


def kernel(token_ids, emb, Wq, Wv, ln_scale, ln_bias, W_role, W_spawn, W_write, W_skip, W_out, b_out, on_scale, on_bias):
    raise NotImplementedError("write your pallas kernel here")


# speedup vs baseline: 18.2142x; 18.2142x over previous
import jax
import jax.numpy as jnp
from jax.experimental import pallas as pl
from jax.experimental.pallas import tpu as pltpu

_VMEM = pl.BlockSpec(memory_space=pltpu.VMEM)
_F32 = jnp.float32


def _ln_rows(h, scale, bias, eps=1e-5):
    mu = jnp.mean(h, axis=-1, keepdims=True)
    var = jnp.mean(jnp.square(h - mu), axis=-1, keepdims=True)
    return (h - mu) * jax.lax.rsqrt(var + eps) * scale + bias


def _gather_kernel(tok_ref, emb_ref, out_ref, sem):
    tpb, nb = out_ref.shape[0], out_ref.shape[1]
    base = pl.program_id(0) * (tpb * nb)
    copies = []
    for mi in range(tpb):
        for b in range(nb):
            tok = tok_ref[base + mi * nb + b]
            cp = pltpu.make_async_copy(emb_ref.at[tok], out_ref.at[mi, b], sem)
            cp.start()
            copies.append(cp)
    for cp in copies:
        cp.wait()


def _rowmm_kernel(x_ref, w_ref, o_ref):
    o_ref[...] = jnp.dot(x_ref[...], w_ref[...],
                         preferred_element_type=_F32)


def _b16(v):
    return v.astype(jnp.bfloat16).astype(_F32)


def _scan_kernel(x0_ref, q0_ref, wv0_ref, ww_ref, lns_ref, lnb_ref,
                 x1_ref, wr_ref, mem_ref):
    t_len, nb, d = x0_ref.shape
    k_ring = mem_ref.shape[0]
    mem_ref[...] = jnp.zeros_like(mem_ref)
    wr_ref[0:k_ring] = jnp.zeros((k_ring, nb, d), _F32)

    def step(t, carry):
        qb = _b16(q0_ref[t])
        memb = _b16(mem_ref[...])
        s = jnp.sum(memb * qb[None], axis=-1)
        mx = jnp.max(s, axis=0, keepdims=True)
        e = jnp.exp(s - mx)
        att = e / jnp.sum(e, axis=0, keepdims=True)
        read = jnp.sum(_b16(att)[:, :, None] * memb, axis=0)
        rv = jnp.dot(read, wv0_ref[...], preferred_element_type=_F32)
        x1 = _ln_rows(x0_ref[t] + rv, lns_ref[...], lnb_ref[...])
        w = jnp.dot(x1, ww_ref[...], preferred_element_type=_F32)
        slot = jax.lax.rem(t, k_ring)
        mem_ref[slot] = w
        x1_ref[t] = x1
        wr_ref[t + k_ring] = w
        return carry

    jax.lax.fori_loop(0, t_len, step, 0)


def _layers_kernel(x1_ref, wr_ref, wq_ref, wv_ref, lns_ref, lnb_ref,
                   wrs0_ref, wrs_ref, wskip_ref, xs_ref, rs_ref,
                   xc_ref, acc_ref):
    i = pl.program_id(1)
    j = pl.program_id(2)
    n_layers_here = pl.num_programs(2)
    tb = x1_ref.shape[1]
    k_ring = wr_ref.shape[1] - (pl.num_programs(1) * tb)
    x1 = x1_ref[0]

    @pl.when(j == 0)
    def _():
        xc_ref[...] = x1
        acc_ref[...] = jnp.dot(x1, wrs0_ref[...], preferred_element_type=_F32)

    x = xc_ref[...]
    q = jnp.dot(x, wq_ref[0], preferred_element_type=_F32)
    wband = wr_ref[0, pl.ds(i * tb, tb + k_ring), :]
    s = jax.lax.dot_general(q, wband, (((1,), (1,)), ((), ())),
                            preferred_element_type=_F32)
    r = jax.lax.broadcasted_iota(jnp.int32, s.shape, 0)
    c = jax.lax.broadcasted_iota(jnp.int32, s.shape, 1)
    band = (c >= r + 1) & (c <= r + k_ring)
    sm = jnp.where(band, s, _F32(-1e30))
    mx = jnp.max(sm, axis=-1, keepdims=True)
    e = jnp.exp(sm - mx)
    att = e / jnp.sum(e, axis=-1, keepdims=True)
    read = jnp.dot(att, wband, preferred_element_type=_F32)
    h = x + jnp.dot(read, wv_ref[0], preferred_element_type=_F32)
    xn = _ln_rows(h, lns_ref[0], lnb_ref[0])
    xc_ref[...] = xn
    acc_ref[...] = acc_ref[...] + jnp.dot(xn, wrs_ref[0],
                                          preferred_element_type=_F32)

    @pl.when(j == n_layers_here - 1)
    def _():
        xs_ref[0] = xn + jnp.dot(x1, wskip_ref[...], preferred_element_type=_F32)
        rs_ref[0] = acc_ref[...] * (1.0 / (n_layers_here + 1))


def _memvec_kernel(wt_ref, wout_ref, bo_ref, ons_ref, onb_ref, ctx_ref):
    k_ring, nb, d = wt_ref.shape
    acc = jnp.zeros((nb, d), _F32) + bo_ref[...]
    for k in range(k_ring):
        acc = acc + jnp.dot(wt_ref[k], wout_ref[pl.ds(k * d, d)],
                            preferred_element_type=_F32)
    ctx_ref[...] = _ln_rows(acc, ons_ref[...], onb_ref[...])


def _logits_kernel(xs_ref, mv_ref, emb_ref, out_ref):
    x = xs_ref[0] + mv_ref[0]
    out_ref[0] = jax.lax.dot_general(x, emb_ref[...], (((1,), (1,)), ((), ())),
                                     preferred_element_type=_F32)


def kernel(token_ids, emb, Wq, Wv, ln_scale, ln_bias, W_role, W_spawn,
           W_write, W_skip, W_out, b_out, on_scale, on_bias):
    nb, t_len = token_ids.shape
    v_size, d = emb.shape
    n_layers = Wq.shape[0]
    k_ring = W_out.shape[0] // d
    nr, ns = W_role.shape[-1], W_spawn.shape[-1]
    nrs = nr + ns

    tok_flat = token_ids.astype(jnp.int32).T.reshape(-1)
    s_sub = d // 128
    emb4 = emb.reshape(v_size, s_sub, 128)
    n_prog = 16 if t_len % 16 == 0 else 1
    x0g = pl.pallas_call(
        _gather_kernel,
        grid=(n_prog,),
        in_specs=[pl.BlockSpec(memory_space=pltpu.SMEM),
                  pl.BlockSpec(memory_space=pl.ANY)],
        out_specs=pl.BlockSpec((t_len // n_prog, nb, s_sub, 128),
                               lambda i: (i, 0, 0, 0)),
        out_shape=jax.ShapeDtypeStruct((t_len, nb, s_sub, 128), _F32),
        scratch_shapes=[pltpu.SemaphoreType.DMA],
        compiler_params=pltpu.CompilerParams(
            dimension_semantics=("parallel",)),
    )(tok_flat, emb4)
    x0 = x0g.reshape(t_len, nb, d)

    x0_2d = x0g.reshape(t_len * nb, d)
    rb = 512 if (t_len * nb) % 512 == 0 else t_len * nb
    q0_2d = pl.pallas_call(
        _rowmm_kernel,
        grid=((t_len * nb) // rb,),
        in_specs=[pl.BlockSpec((rb, d), lambda i: (i, 0)),
                  pl.BlockSpec((d, d), lambda i: (0, 0))],
        out_specs=pl.BlockSpec((rb, d), lambda i: (i, 0)),
        out_shape=jax.ShapeDtypeStruct((t_len * nb, d), _F32),
        compiler_params=pltpu.CompilerParams(
            dimension_semantics=("parallel",)),
    )(x0_2d, Wq[0])
    q0 = q0_2d.reshape(t_len, nb, d)

    x1, wr = pl.pallas_call(
        _scan_kernel,
        in_specs=[_VMEM] * 6,
        out_specs=(_VMEM, _VMEM),
        out_shape=(jax.ShapeDtypeStruct((t_len, nb, d), _F32),
                   jax.ShapeDtypeStruct((t_len + k_ring, nb, d), _F32)),
        scratch_shapes=[pltpu.VMEM((k_ring, nb, d), _F32)],
        compiler_params=pltpu.CompilerParams(
            vmem_limit_bytes=52 * 1024 * 1024),
    )(x0, q0, Wv[0], W_write, ln_scale[0:1], ln_bias[0:1])

    x1b = x1.transpose(1, 0, 2)
    wrb = wr.transpose(1, 0, 2)
    wrs_all = jnp.concatenate([W_role, W_spawn], axis=-1)
    tb = 256 if t_len % 256 == 0 else t_len
    lns3 = ln_scale.reshape(n_layers, 1, d)
    lnb3 = ln_bias.reshape(n_layers, 1, d)
    xs, rs = pl.pallas_call(
        _layers_kernel,
        grid=(nb, t_len // tb, n_layers - 1),
        in_specs=[
            pl.BlockSpec((1, tb, d), lambda b, i, j: (b, i, 0)),
            pl.BlockSpec((1, t_len + k_ring, d), lambda b, i, j: (b, 0, 0)),
            pl.BlockSpec((1, d, d), lambda b, i, j: (j + 1, 0, 0)),
            pl.BlockSpec((1, d, d), lambda b, i, j: (j + 1, 0, 0)),
            pl.BlockSpec((1, 1, d), lambda b, i, j: (j + 1, 0, 0)),
            pl.BlockSpec((1, 1, d), lambda b, i, j: (j + 1, 0, 0)),
            pl.BlockSpec((d, nrs), lambda b, i, j: (0, 0)),
            pl.BlockSpec((1, d, nrs), lambda b, i, j: (j + 1, 0, 0)),
            pl.BlockSpec((d, d), lambda b, i, j: (0, 0)),
        ],
        out_specs=(pl.BlockSpec((1, tb, d), lambda b, i, j: (b, i, 0)),
                   pl.BlockSpec((1, tb, nrs), lambda b, i, j: (b, i, 0))),
        out_shape=(jax.ShapeDtypeStruct((nb, t_len, d), _F32),
                   jax.ShapeDtypeStruct((nb, t_len, nrs), _F32)),
        scratch_shapes=[pltpu.VMEM((tb, d), _F32),
                        pltpu.VMEM((tb, nrs), _F32)],
        compiler_params=pltpu.CompilerParams(
            dimension_semantics=("parallel", "parallel", "arbitrary"),
            vmem_limit_bytes=40 * 1024 * 1024),
    )(x1b, wrb, Wq, Wv, lns3, lnb3, wrs_all[0], wrs_all, W_skip)

    wtail = wr[t_len:]
    ctx = pl.pallas_call(
        _memvec_kernel,
        in_specs=[_VMEM] * 5,
        out_specs=_VMEM,
        out_shape=jax.ShapeDtypeStruct((nb, d), _F32),
        compiler_params=pltpu.CompilerParams(
            vmem_limit_bytes=32 * 1024 * 1024),
    )(wtail, W_out, b_out.reshape(1, d), on_scale.reshape(1, d),
      on_bias.reshape(1, d))

    vb = 3200 if v_size % 3200 == 0 else v_size
    tbl = 256 if t_len % 256 == 0 else t_len
    logits = pl.pallas_call(
        _logits_kernel,
        grid=(v_size // vb, nb, t_len // tbl),
        in_specs=[
            pl.BlockSpec((1, tbl, d), lambda v, b, i: (b, i, 0)),
            pl.BlockSpec((1, 1, d), lambda v, b, i: (b, 0, 0)),
            pl.BlockSpec((vb, d), lambda v, b, i: (v, 0)),
        ],
        out_specs=pl.BlockSpec((1, tbl, vb), lambda v, b, i: (b, i, v)),
        out_shape=jax.ShapeDtypeStruct((nb, t_len, v_size), _F32),
        compiler_params=pltpu.CompilerParams(
            dimension_semantics=("parallel", "arbitrary", "arbitrary"),
            vmem_limit_bytes=48 * 1024 * 1024),
    )(xs, ctx.reshape(nb, 1, d), emb)

    return logits, rs[:, :, :nr], rs[:, :, nr:], ctx


# speedup vs baseline: 18.8510x; 1.0350x over previous
import jax
import jax.numpy as jnp
from jax.experimental import pallas as pl
from jax.experimental.pallas import tpu as pltpu

_VMEM = pl.BlockSpec(memory_space=pltpu.VMEM)
_F32 = jnp.float32


def _ln_rows(h, scale, bias, eps=1e-5):
    mu = jnp.mean(h, axis=-1, keepdims=True)
    var = jnp.mean(jnp.square(h - mu), axis=-1, keepdims=True)
    return (h - mu) * jax.lax.rsqrt(var + eps) * scale + bias


def _gather_kernel(tok_ref, emb_ref, out_ref, sem):
    tpb, nb = out_ref.shape[0], out_ref.shape[1]
    base = pl.program_id(0) * (tpb * nb)
    copies = []
    for mi in range(tpb):
        for b in range(nb):
            tok = tok_ref[base + mi * nb + b]
            cp = pltpu.make_async_copy(emb_ref.at[tok], out_ref.at[mi, b], sem)
            cp.start()
            copies.append(cp)
    for cp in copies:
        cp.wait()


def _rowmm_kernel(x_ref, w_ref, o_ref):
    o_ref[...] = jnp.dot(x_ref[...], w_ref[...],
                         preferred_element_type=_F32)


def _b16(v):
    return v.astype(jnp.bfloat16).astype(_F32)


def _scan_kernel(x0_ref, q0_ref, wv0_ref, ww_ref, lns_ref, lnb_ref,
                 x1_ref, wr_ref, mem_ref):
    t_len, nb, d = x0_ref.shape
    k_ring = mem_ref.shape[0]
    mem_ref[...] = jnp.zeros_like(mem_ref)
    wr_ref[0:k_ring] = jnp.zeros((k_ring, nb, d), _F32)

    def one_step(t, slot):
        qb = _b16(q0_ref[t])
        memb = mem_ref[...]
        s = jnp.sum(memb * qb[None], axis=-1)
        mx = jnp.max(s, axis=0, keepdims=True)
        e = jnp.exp(s - mx)
        att = e / jnp.sum(e, axis=0, keepdims=True)
        read = jnp.sum(_b16(att)[:, :, None] * memb, axis=0)
        rv = jnp.dot(read, wv0_ref[...], preferred_element_type=_F32)
        x1 = _ln_rows(x0_ref[t] + rv, lns_ref[...], lnb_ref[...])
        w = jnp.dot(x1, ww_ref[...], preferred_element_type=_F32)
        mem_ref[slot] = _b16(w)
        x1_ref[t] = x1
        wr_ref[t + k_ring] = w

    if t_len % k_ring == 0:
        def outer(g, carry):
            t0 = g * k_ring
            for r in range(k_ring):
                one_step(t0 + r, r)
            return carry
        jax.lax.fori_loop(0, t_len // k_ring, outer, 0)
    else:
        def step(t, carry):
            one_step(t, jax.lax.rem(t, k_ring))
            return carry
        jax.lax.fori_loop(0, t_len, step, 0)


def _layers_kernel(x1_ref, wr_ref, wq_ref, wv_ref, lns_ref, lnb_ref,
                   wrs0_ref, wrs_ref, wskip_ref, xs_ref, rs_ref,
                   xc_ref, acc_ref):
    i = pl.program_id(1)
    j = pl.program_id(2)
    n_layers_here = pl.num_programs(2)
    tb = x1_ref.shape[1]
    k_ring = wr_ref.shape[1] - (pl.num_programs(1) * tb)
    x1 = x1_ref[0]

    @pl.when(j == 0)
    def _():
        xc_ref[...] = x1
        acc_ref[...] = jnp.dot(x1, wrs0_ref[...], preferred_element_type=_F32)

    x = xc_ref[...]
    q = jnp.dot(x, wq_ref[0], preferred_element_type=_F32)
    wband = wr_ref[0, pl.ds(i * tb, tb + k_ring), :]
    s = jax.lax.dot_general(q, wband, (((1,), (1,)), ((), ())),
                            preferred_element_type=_F32)
    r = jax.lax.broadcasted_iota(jnp.int32, s.shape, 0)
    c = jax.lax.broadcasted_iota(jnp.int32, s.shape, 1)
    band = (c >= r + 1) & (c <= r + k_ring)
    sm = jnp.where(band, s, _F32(-1e30))
    mx = jnp.max(sm, axis=-1, keepdims=True)
    e = jnp.exp(sm - mx)
    att = e / jnp.sum(e, axis=-1, keepdims=True)
    read = jnp.dot(att, wband, preferred_element_type=_F32)
    h = x + jnp.dot(read, wv_ref[0], preferred_element_type=_F32)
    xn = _ln_rows(h, lns_ref[0], lnb_ref[0])
    xc_ref[...] = xn
    acc_ref[...] = acc_ref[...] + jnp.dot(xn, wrs_ref[0],
                                          preferred_element_type=_F32)

    @pl.when(j == n_layers_here - 1)
    def _():
        xs_ref[0] = xn + jnp.dot(x1, wskip_ref[...], preferred_element_type=_F32)
        rs_ref[0] = acc_ref[...] * (1.0 / (n_layers_here + 1))


def _memvec_kernel(wt_ref, wout_ref, bo_ref, ons_ref, onb_ref, ctx_ref):
    k_ring, nb, d = wt_ref.shape
    acc = jnp.zeros((nb, d), _F32) + bo_ref[...]
    for k in range(k_ring):
        acc = acc + jnp.dot(wt_ref[k], wout_ref[pl.ds(k * d, d)],
                            preferred_element_type=_F32)
    ctx_ref[...] = _ln_rows(acc, ons_ref[...], onb_ref[...])


def _logits_kernel(xs_ref, mv_ref, emb_ref, out_ref):
    x = xs_ref[0] + mv_ref[0]
    out_ref[0] = jax.lax.dot_general(x, emb_ref[...], (((1,), (1,)), ((), ())),
                                     preferred_element_type=_F32)


def kernel(token_ids, emb, Wq, Wv, ln_scale, ln_bias, W_role, W_spawn,
           W_write, W_skip, W_out, b_out, on_scale, on_bias):
    nb, t_len = token_ids.shape
    v_size, d = emb.shape
    n_layers = Wq.shape[0]
    k_ring = W_out.shape[0] // d
    nr, ns = W_role.shape[-1], W_spawn.shape[-1]
    nrs = nr + ns

    tok_flat = token_ids.astype(jnp.int32).T.reshape(-1)
    s_sub = d // 128
    emb4 = emb.reshape(v_size, s_sub, 128)
    n_prog = 16 if t_len % 16 == 0 else 1
    x0g = pl.pallas_call(
        _gather_kernel,
        grid=(n_prog,),
        in_specs=[pl.BlockSpec(memory_space=pltpu.SMEM),
                  pl.BlockSpec(memory_space=pl.ANY)],
        out_specs=pl.BlockSpec((t_len // n_prog, nb, s_sub, 128),
                               lambda i: (i, 0, 0, 0)),
        out_shape=jax.ShapeDtypeStruct((t_len, nb, s_sub, 128), _F32),
        scratch_shapes=[pltpu.SemaphoreType.DMA],
        compiler_params=pltpu.CompilerParams(
            dimension_semantics=("parallel",)),
    )(tok_flat, emb4)
    x0 = x0g.reshape(t_len, nb, d)

    x0_2d = x0g.reshape(t_len * nb, d)
    rb = 512 if (t_len * nb) % 512 == 0 else t_len * nb
    q0_2d = pl.pallas_call(
        _rowmm_kernel,
        grid=((t_len * nb) // rb,),
        in_specs=[pl.BlockSpec((rb, d), lambda i: (i, 0)),
                  pl.BlockSpec((d, d), lambda i: (0, 0))],
        out_specs=pl.BlockSpec((rb, d), lambda i: (i, 0)),
        out_shape=jax.ShapeDtypeStruct((t_len * nb, d), _F32),
        compiler_params=pltpu.CompilerParams(
            dimension_semantics=("parallel",)),
    )(x0_2d, Wq[0])
    q0 = q0_2d.reshape(t_len, nb, d)

    x1, wr = pl.pallas_call(
        _scan_kernel,
        in_specs=[_VMEM] * 6,
        out_specs=(_VMEM, _VMEM),
        out_shape=(jax.ShapeDtypeStruct((t_len, nb, d), _F32),
                   jax.ShapeDtypeStruct((t_len + k_ring, nb, d), _F32)),
        scratch_shapes=[pltpu.VMEM((k_ring, nb, d), _F32)],
        compiler_params=pltpu.CompilerParams(
            vmem_limit_bytes=52 * 1024 * 1024),
    )(x0, q0, Wv[0], W_write, ln_scale[0:1], ln_bias[0:1])

    x1b = x1.transpose(1, 0, 2)
    wrb = wr.transpose(1, 0, 2)
    wrs_all = jnp.concatenate([W_role, W_spawn], axis=-1)
    tb = 256 if t_len % 256 == 0 else t_len
    lns3 = ln_scale.reshape(n_layers, 1, d)
    lnb3 = ln_bias.reshape(n_layers, 1, d)
    xs, rs = pl.pallas_call(
        _layers_kernel,
        grid=(nb, t_len // tb, n_layers - 1),
        in_specs=[
            pl.BlockSpec((1, tb, d), lambda b, i, j: (b, i, 0)),
            pl.BlockSpec((1, t_len + k_ring, d), lambda b, i, j: (b, 0, 0)),
            pl.BlockSpec((1, d, d), lambda b, i, j: (j + 1, 0, 0)),
            pl.BlockSpec((1, d, d), lambda b, i, j: (j + 1, 0, 0)),
            pl.BlockSpec((1, 1, d), lambda b, i, j: (j + 1, 0, 0)),
            pl.BlockSpec((1, 1, d), lambda b, i, j: (j + 1, 0, 0)),
            pl.BlockSpec((d, nrs), lambda b, i, j: (0, 0)),
            pl.BlockSpec((1, d, nrs), lambda b, i, j: (j + 1, 0, 0)),
            pl.BlockSpec((d, d), lambda b, i, j: (0, 0)),
        ],
        out_specs=(pl.BlockSpec((1, tb, d), lambda b, i, j: (b, i, 0)),
                   pl.BlockSpec((1, tb, nrs), lambda b, i, j: (b, i, 0))),
        out_shape=(jax.ShapeDtypeStruct((nb, t_len, d), _F32),
                   jax.ShapeDtypeStruct((nb, t_len, nrs), _F32)),
        scratch_shapes=[pltpu.VMEM((tb, d), _F32),
                        pltpu.VMEM((tb, nrs), _F32)],
        compiler_params=pltpu.CompilerParams(
            dimension_semantics=("parallel", "parallel", "arbitrary"),
            vmem_limit_bytes=40 * 1024 * 1024),
    )(x1b, wrb, Wq, Wv, lns3, lnb3, wrs_all[0], wrs_all, W_skip)

    wtail = wr[t_len:]
    ctx = pl.pallas_call(
        _memvec_kernel,
        in_specs=[_VMEM] * 5,
        out_specs=_VMEM,
        out_shape=jax.ShapeDtypeStruct((nb, d), _F32),
        compiler_params=pltpu.CompilerParams(
            vmem_limit_bytes=32 * 1024 * 1024),
    )(wtail, W_out, b_out.reshape(1, d), on_scale.reshape(1, d),
      on_bias.reshape(1, d))

    vb = 3200 if v_size % 3200 == 0 else v_size
    tbl = 256 if t_len % 256 == 0 else t_len
    logits = pl.pallas_call(
        _logits_kernel,
        grid=(v_size // vb, nb, t_len // tbl),
        in_specs=[
            pl.BlockSpec((1, tbl, d), lambda v, b, i: (b, i, 0)),
            pl.BlockSpec((1, 1, d), lambda v, b, i: (b, 0, 0)),
            pl.BlockSpec((vb, d), lambda v, b, i: (v, 0)),
        ],
        out_specs=pl.BlockSpec((1, tbl, vb), lambda v, b, i: (b, i, v)),
        out_shape=jax.ShapeDtypeStruct((nb, t_len, v_size), _F32),
        compiler_params=pltpu.CompilerParams(
            dimension_semantics=("parallel", "arbitrary", "arbitrary"),
            vmem_limit_bytes=48 * 1024 * 1024),
    )(xs, ctx.reshape(nb, 1, d), emb)

    return logits, rs[:, :, :nr], rs[:, :, nr:], ctx


# speedup vs baseline: 19.5433x; 1.0367x over previous
import jax
import jax.numpy as jnp
from jax.experimental import pallas as pl
from jax.experimental.pallas import tpu as pltpu

_VMEM = pl.BlockSpec(memory_space=pltpu.VMEM)
_F32 = jnp.float32


def _ln_rows(h, scale, bias, eps=1e-5):
    mu = jnp.mean(h, axis=-1, keepdims=True)
    var = jnp.mean(jnp.square(h - mu), axis=-1, keepdims=True)
    return (h - mu) * jax.lax.rsqrt(var + eps) * scale + bias


def _gather_kernel(tok_ref, emb_ref, out_ref, sem):
    tpb, nb = out_ref.shape[0], out_ref.shape[1]
    base = pl.program_id(0) * (tpb * nb)
    copies = []
    for mi in range(tpb):
        for b in range(nb):
            tok = tok_ref[base + mi * nb + b]
            cp = pltpu.make_async_copy(emb_ref.at[tok], out_ref.at[mi, b], sem)
            cp.start()
            copies.append(cp)
    for cp in copies:
        cp.wait()


def _rowmm_kernel(x_ref, w_ref, o_ref):
    o_ref[...] = jnp.dot(x_ref[...], w_ref[...],
                         preferred_element_type=_F32)


def _b16(v):
    return v.astype(jnp.bfloat16).astype(_F32)


def _scan_kernel(x0_ref, q0_ref, wv0_ref, ww_ref, lns_ref, lnb_ref,
                 x1_ref, wr_ref, mem_ref):
    t_len, nb, d = x0_ref.shape
    k_ring = mem_ref.shape[0]
    mem_ref[...] = jnp.zeros_like(mem_ref)
    wr_ref[0:k_ring] = jnp.zeros((k_ring, nb, d), _F32)

    def one_step(t, slot):
        qb = _b16(q0_ref[t])
        memb = mem_ref[...]
        s = jnp.sum(memb * qb[None], axis=-1)
        mx = jnp.max(s, axis=0, keepdims=True)
        e = jnp.exp(s - mx)
        att = e / jnp.sum(e, axis=0, keepdims=True)
        read = jnp.sum(_b16(att)[:, :, None] * memb, axis=0)
        rv = jnp.dot(read, wv0_ref[...], preferred_element_type=_F32)
        x1 = _ln_rows(x0_ref[t] + rv, lns_ref[...], lnb_ref[...])
        w = jnp.dot(x1, ww_ref[...], preferred_element_type=_F32)
        mem_ref[slot] = _b16(w)
        x1_ref[t] = x1
        wr_ref[t + k_ring] = w

    if t_len % k_ring == 0:
        def outer(g, carry):
            t0 = g * k_ring
            for r in range(k_ring):
                one_step(t0 + r, r)
            return carry
        jax.lax.fori_loop(0, t_len // k_ring, outer, 0)
    else:
        def step(t, carry):
            one_step(t, jax.lax.rem(t, k_ring))
            return carry
        jax.lax.fori_loop(0, t_len, step, 0)


def _layers_kernel(x1_ref, wr_ref, wq_ref, wv_ref, lns_ref, lnb_ref,
                   wrs0_ref, wrs_ref, wskip_ref, xs_ref, rs_ref,
                   xc_ref, acc_ref):
    i = pl.program_id(1)
    j = pl.program_id(2)
    n_layers_here = pl.num_programs(2)
    tb = x1_ref.shape[1]
    k_ring = wr_ref.shape[1] - (pl.num_programs(1) * tb)
    x1 = x1_ref[0]

    @pl.when(j == 0)
    def _():
        xc_ref[...] = x1
        acc_ref[...] = jnp.dot(x1, wrs0_ref[...], preferred_element_type=_F32)

    x = xc_ref[...]
    q = jnp.dot(x, wq_ref[0], preferred_element_type=_F32)
    wband = wr_ref[0, pl.ds(i * tb, tb + k_ring), :]
    s = jax.lax.dot_general(q, wband, (((1,), (1,)), ((), ())),
                            preferred_element_type=_F32)
    r = jax.lax.broadcasted_iota(jnp.int32, s.shape, 0)
    c = jax.lax.broadcasted_iota(jnp.int32, s.shape, 1)
    band = (c >= r + 1) & (c <= r + k_ring)
    sm = jnp.where(band, s, _F32(-1e30))
    mx = jnp.max(sm, axis=-1, keepdims=True)
    e = jnp.exp(sm - mx)
    att = e / jnp.sum(e, axis=-1, keepdims=True)
    read = jnp.dot(att, wband, preferred_element_type=_F32)
    h = x + jnp.dot(read, wv_ref[0], preferred_element_type=_F32)
    xn = _ln_rows(h, lns_ref[0], lnb_ref[0])
    xc_ref[...] = xn
    acc_ref[...] = acc_ref[...] + jnp.dot(xn, wrs_ref[0],
                                          preferred_element_type=_F32)

    @pl.when(j == n_layers_here - 1)
    def _():
        xs_ref[0] = xn + jnp.dot(x1, wskip_ref[...], preferred_element_type=_F32)
        rs_ref[0] = acc_ref[...] * (1.0 / (n_layers_here + 1))


def _memvec_kernel(wt_ref, wout_ref, bo_ref, ons_ref, onb_ref, ctx_ref):
    k_ring, nb, d = wt_ref.shape
    acc = jnp.zeros((nb, d), _F32) + bo_ref[...]
    for k in range(k_ring):
        acc = acc + jnp.dot(wt_ref[k], wout_ref[pl.ds(k * d, d)],
                            preferred_element_type=_F32)
    ctx_ref[...] = _ln_rows(acc, ons_ref[...], onb_ref[...])


def _logits_kernel(xs_ref, mv_ref, emb_ref, out_ref):
    x = xs_ref[0] + mv_ref[0]
    out_ref[0] = jax.lax.dot_general(x, emb_ref[...], (((1,), (1,)), ((), ())),
                                     preferred_element_type=_F32)


def kernel(token_ids, emb, Wq, Wv, ln_scale, ln_bias, W_role, W_spawn,
           W_write, W_skip, W_out, b_out, on_scale, on_bias):
    nb, t_len = token_ids.shape
    v_size, d = emb.shape
    n_layers = Wq.shape[0]
    k_ring = W_out.shape[0] // d
    nr, ns = W_role.shape[-1], W_spawn.shape[-1]
    nrs = nr + ns

    tok_flat = token_ids.astype(jnp.int32).T.reshape(-1)
    s_sub = d // 128
    emb4 = emb.reshape(v_size, s_sub, 128)
    n_prog = 16 if t_len % 16 == 0 else 1
    x0g = pl.pallas_call(
        _gather_kernel,
        grid=(n_prog,),
        in_specs=[pl.BlockSpec(memory_space=pltpu.SMEM),
                  pl.BlockSpec(memory_space=pl.ANY)],
        out_specs=pl.BlockSpec((t_len // n_prog, nb, s_sub, 128),
                               lambda i: (i, 0, 0, 0)),
        out_shape=jax.ShapeDtypeStruct((t_len, nb, s_sub, 128), _F32),
        scratch_shapes=[pltpu.SemaphoreType.DMA],
        compiler_params=pltpu.CompilerParams(
            dimension_semantics=("parallel",)),
    )(tok_flat, emb4)
    x0 = x0g.reshape(t_len, nb, d)

    x0_2d = x0g.reshape(t_len * nb, d)
    rb = 512 if (t_len * nb) % 512 == 0 else t_len * nb
    q0_2d = pl.pallas_call(
        _rowmm_kernel,
        grid=((t_len * nb) // rb,),
        in_specs=[pl.BlockSpec((rb, d), lambda i: (i, 0)),
                  pl.BlockSpec((d, d), lambda i: (0, 0))],
        out_specs=pl.BlockSpec((rb, d), lambda i: (i, 0)),
        out_shape=jax.ShapeDtypeStruct((t_len * nb, d), _F32),
        compiler_params=pltpu.CompilerParams(
            dimension_semantics=("parallel",)),
    )(x0_2d, Wq[0])
    q0 = q0_2d.reshape(t_len, nb, d)

    x1, wr = pl.pallas_call(
        _scan_kernel,
        in_specs=[_VMEM] * 6,
        out_specs=(_VMEM, _VMEM),
        out_shape=(jax.ShapeDtypeStruct((t_len, nb, d), _F32),
                   jax.ShapeDtypeStruct((t_len + k_ring, nb, d), _F32)),
        scratch_shapes=[pltpu.VMEM((k_ring, nb, d), _F32)],
        compiler_params=pltpu.CompilerParams(
            vmem_limit_bytes=52 * 1024 * 1024),
    )(x0, q0, Wv[0], W_write, ln_scale[0:1], ln_bias[0:1])

    x1b = x1.transpose(1, 0, 2)
    wrb = wr.transpose(1, 0, 2)
    wrs_all = jnp.concatenate([W_role, W_spawn], axis=-1)
    tb = 256 if t_len % 256 == 0 else t_len
    lns3 = ln_scale.reshape(n_layers, 1, d)
    lnb3 = ln_bias.reshape(n_layers, 1, d)
    xs, rs = pl.pallas_call(
        _layers_kernel,
        grid=(nb, t_len // tb, n_layers - 1),
        in_specs=[
            pl.BlockSpec((1, tb, d), lambda b, i, j: (b, i, 0)),
            pl.BlockSpec((1, t_len + k_ring, d), lambda b, i, j: (b, 0, 0)),
            pl.BlockSpec((1, d, d), lambda b, i, j: (j + 1, 0, 0)),
            pl.BlockSpec((1, d, d), lambda b, i, j: (j + 1, 0, 0)),
            pl.BlockSpec((1, 1, d), lambda b, i, j: (j + 1, 0, 0)),
            pl.BlockSpec((1, 1, d), lambda b, i, j: (j + 1, 0, 0)),
            pl.BlockSpec((d, nrs), lambda b, i, j: (0, 0)),
            pl.BlockSpec((1, d, nrs), lambda b, i, j: (j + 1, 0, 0)),
            pl.BlockSpec((d, d), lambda b, i, j: (0, 0)),
        ],
        out_specs=(pl.BlockSpec((1, tb, d), lambda b, i, j: (b, i, 0)),
                   pl.BlockSpec((1, tb, nrs), lambda b, i, j: (b, i, 0))),
        out_shape=(jax.ShapeDtypeStruct((nb, t_len, d), _F32),
                   jax.ShapeDtypeStruct((nb, t_len, nrs), _F32)),
        scratch_shapes=[pltpu.VMEM((tb, d), _F32),
                        pltpu.VMEM((tb, nrs), _F32)],
        compiler_params=pltpu.CompilerParams(
            dimension_semantics=("parallel", "parallel", "arbitrary"),
            vmem_limit_bytes=40 * 1024 * 1024),
    )(x1b, wrb, Wq, Wv, lns3, lnb3, wrs_all[0], wrs_all, W_skip)

    wtail = wr[t_len:]
    ctx = pl.pallas_call(
        _memvec_kernel,
        in_specs=[_VMEM] * 5,
        out_specs=_VMEM,
        out_shape=jax.ShapeDtypeStruct((nb, d), _F32),
        compiler_params=pltpu.CompilerParams(
            vmem_limit_bytes=32 * 1024 * 1024),
    )(wtail, W_out, b_out.reshape(1, d), on_scale.reshape(1, d),
      on_bias.reshape(1, d))

    vb = 3200 if v_size % 3200 == 0 else v_size
    tbl = 512 if t_len % 512 == 0 else t_len
    logits = pl.pallas_call(
        _logits_kernel,
        grid=(v_size // vb, nb, t_len // tbl),
        in_specs=[
            pl.BlockSpec((1, tbl, d), lambda v, b, i: (b, i, 0)),
            pl.BlockSpec((1, 1, d), lambda v, b, i: (b, 0, 0)),
            pl.BlockSpec((vb, d), lambda v, b, i: (v, 0)),
        ],
        out_specs=pl.BlockSpec((1, tbl, vb), lambda v, b, i: (b, i, v)),
        out_shape=jax.ShapeDtypeStruct((nb, t_len, v_size), _F32),
        compiler_params=pltpu.CompilerParams(
            dimension_semantics=("parallel", "arbitrary", "arbitrary"),
            vmem_limit_bytes=48 * 1024 * 1024),
    )(xs, ctx.reshape(nb, 1, d), emb)

    return logits, rs[:, :, :nr], rs[:, :, nr:], ctx


# speedup vs baseline: 20.8015x; 1.0644x over previous
import jax
import jax.numpy as jnp
from jax.experimental import pallas as pl
from jax.experimental.pallas import tpu as pltpu

_VMEM = pl.BlockSpec(memory_space=pltpu.VMEM)
_F32 = jnp.float32


def _ln_rows(h, scale, bias, eps=1e-5):
    mu = jnp.mean(h, axis=-1, keepdims=True)
    var = jnp.mean(jnp.square(h - mu), axis=-1, keepdims=True)
    return (h - mu) * jax.lax.rsqrt(var + eps) * scale + bias


def _gather_kernel(tok_ref, emb_ref, out_ref, sem):
    tpb, nb = out_ref.shape[0], out_ref.shape[1]
    base = pl.program_id(0) * (tpb * nb)
    copies = []
    for mi in range(tpb):
        for b in range(nb):
            tok = tok_ref[base + mi * nb + b]
            cp = pltpu.make_async_copy(emb_ref.at[tok], out_ref.at[mi, b], sem)
            cp.start()
            copies.append(cp)
    for cp in copies:
        cp.wait()


def _rowmm_kernel(x_ref, w_ref, o_ref):
    o_ref[...] = jnp.dot(x_ref[...], w_ref[...],
                         preferred_element_type=_F32)


def _b16(v):
    return v.astype(jnp.bfloat16).astype(_F32)


def _scan_kernel(x0_ref, q0_ref, wv0_ref, ww_ref, lns_ref, lnb_ref,
                 x1_ref, wr_ref, mem_ref):
    t_len, nb, d = x0_ref.shape
    k_ring = mem_ref.shape[0]
    mem_ref[...] = jnp.zeros_like(mem_ref)
    wr_ref[0:k_ring] = jnp.zeros((k_ring, nb, d), _F32)

    def one_step(t, slot):
        qb = _b16(q0_ref[t])
        memb = mem_ref[...]
        s = jnp.sum(memb * qb[None], axis=-1)
        mx = jnp.max(s, axis=0, keepdims=True)
        e = jnp.exp(s - mx)
        att = e / jnp.sum(e, axis=0, keepdims=True)
        read = jnp.sum(_b16(att)[:, :, None] * memb, axis=0)
        rv = jnp.dot(read, wv0_ref[...], preferred_element_type=_F32)
        h = x0_ref[t] + rv
        mu = jnp.mean(h, axis=-1, keepdims=True)
        ms = jnp.mean(h * h, axis=-1, keepdims=True)
        var = ms - mu * mu
        x1 = (h - mu) * jax.lax.rsqrt(var + 1e-5) * lns_ref[...] + lnb_ref[...]
        w = jnp.dot(x1, ww_ref[...], preferred_element_type=_F32)
        mem_ref[slot] = _b16(w)
        x1_ref[t] = x1
        wr_ref[t + k_ring] = w

    if t_len % k_ring == 0:
        def outer(g, carry):
            t0 = g * k_ring
            for r in range(k_ring):
                one_step(t0 + r, r)
            return carry
        jax.lax.fori_loop(0, t_len // k_ring, outer, 0)
    else:
        def step(t, carry):
            one_step(t, jax.lax.rem(t, k_ring))
            return carry
        jax.lax.fori_loop(0, t_len, step, 0)


def _layers_kernel(x1_ref, wr_ref, wq_ref, wv_ref, lns_ref, lnb_ref,
                   wrs0_ref, wrs_ref, wskip_ref, xs_ref, rs_ref,
                   xc_ref, acc_ref):
    i = pl.program_id(1)
    j = pl.program_id(2)
    n_layers_here = pl.num_programs(2)
    tb = x1_ref.shape[1]
    k_ring = wr_ref.shape[1] - (pl.num_programs(1) * tb)
    x1 = x1_ref[0]

    @pl.when(j == 0)
    def _():
        xc_ref[...] = x1
        acc_ref[...] = jnp.dot(x1, wrs0_ref[...], preferred_element_type=_F32)

    x = xc_ref[...]
    q = jnp.dot(x, wq_ref[0], preferred_element_type=_F32)
    wband = wr_ref[0, pl.ds(i * tb, tb + k_ring), :]
    s = jax.lax.dot_general(q, wband, (((1,), (1,)), ((), ())),
                            preferred_element_type=_F32)
    r = jax.lax.broadcasted_iota(jnp.int32, s.shape, 0)
    c = jax.lax.broadcasted_iota(jnp.int32, s.shape, 1)
    band = (c >= r + 1) & (c <= r + k_ring)
    sm = jnp.where(band, s, _F32(-1e30))
    mx = jnp.max(sm, axis=-1, keepdims=True)
    e = jnp.exp(sm - mx)
    att = e / jnp.sum(e, axis=-1, keepdims=True)
    read = jnp.dot(att, wband, preferred_element_type=_F32)
    h = x + jnp.dot(read, wv_ref[0], preferred_element_type=_F32)
    xn = _ln_rows(h, lns_ref[0], lnb_ref[0])
    xc_ref[...] = xn
    acc_ref[...] = acc_ref[...] + jnp.dot(xn, wrs_ref[0],
                                          preferred_element_type=_F32)

    @pl.when(j == n_layers_here - 1)
    def _():
        xs_ref[0] = xn + jnp.dot(x1, wskip_ref[...], preferred_element_type=_F32)
        rs_ref[0] = acc_ref[...] * (1.0 / (n_layers_here + 1))


def _memvec_kernel(wt_ref, wout_ref, bo_ref, ons_ref, onb_ref, ctx_ref):
    k_ring, nb, d = wt_ref.shape
    acc = jnp.zeros((nb, d), _F32) + bo_ref[...]
    for k in range(k_ring):
        acc = acc + jnp.dot(wt_ref[k], wout_ref[pl.ds(k * d, d)],
                            preferred_element_type=_F32)
    ctx_ref[...] = _ln_rows(acc, ons_ref[...], onb_ref[...])


def _logits_kernel(xs_ref, mv_ref, emb_ref, out_ref):
    x = xs_ref[0] + mv_ref[0]
    out_ref[0] = jax.lax.dot_general(x, emb_ref[...], (((1,), (1,)), ((), ())),
                                     preferred_element_type=_F32)


def kernel(token_ids, emb, Wq, Wv, ln_scale, ln_bias, W_role, W_spawn,
           W_write, W_skip, W_out, b_out, on_scale, on_bias):
    nb, t_len = token_ids.shape
    v_size, d = emb.shape
    n_layers = Wq.shape[0]
    k_ring = W_out.shape[0] // d
    nr, ns = W_role.shape[-1], W_spawn.shape[-1]
    nrs = nr + ns

    tok_flat = token_ids.astype(jnp.int32).T.reshape(-1)
    s_sub = d // 128
    emb4 = emb.reshape(v_size, s_sub, 128)
    n_prog = 16 if t_len % 16 == 0 else 1
    x0g = pl.pallas_call(
        _gather_kernel,
        grid=(n_prog,),
        in_specs=[pl.BlockSpec(memory_space=pltpu.SMEM),
                  pl.BlockSpec(memory_space=pl.ANY)],
        out_specs=pl.BlockSpec((t_len // n_prog, nb, s_sub, 128),
                               lambda i: (i, 0, 0, 0)),
        out_shape=jax.ShapeDtypeStruct((t_len, nb, s_sub, 128), _F32),
        scratch_shapes=[pltpu.SemaphoreType.DMA],
        compiler_params=pltpu.CompilerParams(
            dimension_semantics=("parallel",)),
    )(tok_flat, emb4)
    x0 = x0g.reshape(t_len, nb, d)

    x0_2d = x0g.reshape(t_len * nb, d)
    rb = 512 if (t_len * nb) % 512 == 0 else t_len * nb
    q0_2d = pl.pallas_call(
        _rowmm_kernel,
        grid=((t_len * nb) // rb,),
        in_specs=[pl.BlockSpec((rb, d), lambda i: (i, 0)),
                  pl.BlockSpec((d, d), lambda i: (0, 0))],
        out_specs=pl.BlockSpec((rb, d), lambda i: (i, 0)),
        out_shape=jax.ShapeDtypeStruct((t_len * nb, d), _F32),
        compiler_params=pltpu.CompilerParams(
            dimension_semantics=("parallel",)),
    )(x0_2d, Wq[0])
    q0 = q0_2d.reshape(t_len, nb, d)

    x1, wr = pl.pallas_call(
        _scan_kernel,
        in_specs=[_VMEM] * 6,
        out_specs=(_VMEM, _VMEM),
        out_shape=(jax.ShapeDtypeStruct((t_len, nb, d), _F32),
                   jax.ShapeDtypeStruct((t_len + k_ring, nb, d), _F32)),
        scratch_shapes=[pltpu.VMEM((k_ring, nb, d), _F32)],
        compiler_params=pltpu.CompilerParams(
            vmem_limit_bytes=52 * 1024 * 1024),
    )(x0, q0, Wv[0], W_write, ln_scale[0:1], ln_bias[0:1])

    x1b = x1.transpose(1, 0, 2)
    wrb = wr.transpose(1, 0, 2)
    wrs_all = jnp.concatenate([W_role, W_spawn], axis=-1)
    tb = 256 if t_len % 256 == 0 else t_len
    lns3 = ln_scale.reshape(n_layers, 1, d)
    lnb3 = ln_bias.reshape(n_layers, 1, d)
    xs, rs = pl.pallas_call(
        _layers_kernel,
        grid=(nb, t_len // tb, n_layers - 1),
        in_specs=[
            pl.BlockSpec((1, tb, d), lambda b, i, j: (b, i, 0)),
            pl.BlockSpec((1, t_len + k_ring, d), lambda b, i, j: (b, 0, 0)),
            pl.BlockSpec((1, d, d), lambda b, i, j: (j + 1, 0, 0)),
            pl.BlockSpec((1, d, d), lambda b, i, j: (j + 1, 0, 0)),
            pl.BlockSpec((1, 1, d), lambda b, i, j: (j + 1, 0, 0)),
            pl.BlockSpec((1, 1, d), lambda b, i, j: (j + 1, 0, 0)),
            pl.BlockSpec((d, nrs), lambda b, i, j: (0, 0)),
            pl.BlockSpec((1, d, nrs), lambda b, i, j: (j + 1, 0, 0)),
            pl.BlockSpec((d, d), lambda b, i, j: (0, 0)),
        ],
        out_specs=(pl.BlockSpec((1, tb, d), lambda b, i, j: (b, i, 0)),
                   pl.BlockSpec((1, tb, nrs), lambda b, i, j: (b, i, 0))),
        out_shape=(jax.ShapeDtypeStruct((nb, t_len, d), _F32),
                   jax.ShapeDtypeStruct((nb, t_len, nrs), _F32)),
        scratch_shapes=[pltpu.VMEM((tb, d), _F32),
                        pltpu.VMEM((tb, nrs), _F32)],
        compiler_params=pltpu.CompilerParams(
            dimension_semantics=("parallel", "parallel", "arbitrary"),
            vmem_limit_bytes=40 * 1024 * 1024),
    )(x1b, wrb, Wq, Wv, lns3, lnb3, wrs_all[0], wrs_all, W_skip)

    wtail = wr[t_len:]
    ctx = pl.pallas_call(
        _memvec_kernel,
        in_specs=[_VMEM] * 5,
        out_specs=_VMEM,
        out_shape=jax.ShapeDtypeStruct((nb, d), _F32),
        compiler_params=pltpu.CompilerParams(
            vmem_limit_bytes=32 * 1024 * 1024),
    )(wtail, W_out, b_out.reshape(1, d), on_scale.reshape(1, d),
      on_bias.reshape(1, d))

    vb = 3200 if v_size % 3200 == 0 else v_size
    tbl = 512 if t_len % 512 == 0 else t_len
    logits = pl.pallas_call(
        _logits_kernel,
        grid=(v_size // vb, nb, t_len // tbl),
        in_specs=[
            pl.BlockSpec((1, tbl, d), lambda v, b, i: (b, i, 0)),
            pl.BlockSpec((1, 1, d), lambda v, b, i: (b, 0, 0)),
            pl.BlockSpec((vb, d), lambda v, b, i: (v, 0)),
        ],
        out_specs=pl.BlockSpec((1, tbl, vb), lambda v, b, i: (b, i, v)),
        out_shape=jax.ShapeDtypeStruct((nb, t_len, v_size), _F32),
        compiler_params=pltpu.CompilerParams(
            dimension_semantics=("parallel", "arbitrary", "arbitrary"),
            vmem_limit_bytes=48 * 1024 * 1024),
    )(xs, ctx.reshape(nb, 1, d), emb)

    return logits, rs[:, :, :nr], rs[:, :, nr:], ctx
